# Initial kernel scaffold; baseline (speedup 1.0000x reference)
import jax
import jax.numpy as jnp
from jax.experimental import pallas as pl


def kernel(x, edge_index, edge_attr, batch, W_neigh1, W_root1, b1, W_neigh2, W_root2, b2, W_neigh3, W_root3, b3, W_fc, b_fc):
    raise NotImplementedError("write your pallas kernel here")


# speedup vs baseline: 3.8261x; 3.8261x over previous
import functools

import jax
import jax.numpy as jnp
from jax import lax
from jax.experimental import pallas as pl
from jax.experimental.pallas import tpu as pltpu
from jax.experimental.pallas import tpu_sc as plsc

F32 = jnp.float32
CH = 128
DW = 16


def _build_sc_agg(n, e, f):
    ns = 16
    fh = f // 2
    nchunk = e // CH
    nb = nchunk // ns
    tail = nchunk - nb * ns
    rpt = (n // ns) // 8 * 8
    rem = n - rpt * ns

    out_type = (jax.ShapeDtypeStruct((n, fh), F32),
                jax.ShapeDtypeStruct((n, fh), F32),
                jax.ShapeDtypeStruct((n, DW), F32))
    scratch = [pltpu.VMEM_SHARED((n, fh), F32),
               pltpu.VMEM_SHARED((n, DW), F32),
               pltpu.VMEM((CH,), jnp.int32),
               pltpu.VMEM((CH,), jnp.int32),
               pltpu.VMEM((CH, fh), F32),
               pltpu.VMEM((CH, DW), F32),
               pltpu.VMEM((CH, DW), F32),
               pltpu.SemaphoreType.DMA]
    mesh = plsc.VectorSubcoreMesh(core_axis_name="c", subcore_axis_name="s")

    def body(ya_hbm, yb_hbm, src_hbm, dst_hbm, z_hbm, z16_hbm, ones_hbm,
             pa_out, pb_out, d_out,
             acc_sh, deg_sh, src_v, dst_v, rows_v, ones_v, z16_v, sem):
        c = lax.axis_index("c")
        s = lax.axis_index("s")

        def staged(src_ref, dst_ref, row0, nrows, stage):
            off = 0
            while off < nrows:
                step = min(CH, nrows - off)
                r = pl.ds(pl.multiple_of(row0 + off, 8), step)
                pltpu.sync_copy(src_ref.at[r], stage.at[pl.ds(0, step)])
                pltpu.sync_copy(stage.at[pl.ds(0, step)], dst_ref.at[r])
                off += step

        def spread(fn):
            fn(pl.multiple_of(s * rpt, 8), rpt)
            if rem:
                @pl.when(s == ns - 1)
                def _():
                    fn(rpt * ns, rem)

        spread(lambda r0, nr: staged(z_hbm, acc_sh, r0, nr, rows_v))

        @pl.when(c == 0)
        def _():
            spread(lambda r0, nr: staged(z16_hbm, deg_sh, r0, nr, z16_v))
            pltpu.sync_copy(ones_hbm, ones_v)
        plsc.subcore_barrier()

        def chunk(k, tbl, dodeg):
            off = pl.multiple_of(k * CH, CH)
            pltpu.sync_copy(src_hbm.at[pl.ds(off, CH)], src_v)
            pltpu.sync_copy(dst_hbm.at[pl.ds(off, CH)], dst_v)
            pltpu.async_copy(tbl.at[src_v], rows_v, sem).wait()
            pltpu.sync_copy(rows_v, acc_sh.at[dst_v], add=True)
            if dodeg:
                pltpu.sync_copy(ones_v, deg_sh.at[dst_v], add=True)

        def edge_loop(tbl, dodeg):
            def loop_body(i, carry):
                chunk(s + i * ns, tbl, dodeg)
                return carry
            lax.fori_loop(0, nb, loop_body, 0)
            if tail:
                @pl.when(s < tail)
                def _():
                    chunk(nb * ns + s, tbl, dodeg)

        @pl.when(c == 0)
        def _():
            edge_loop(ya_hbm, True)

        @pl.when(c == 1)
        def _():
            edge_loop(yb_hbm, False)

        plsc.subcore_barrier()

        @pl.when(c == 0)
        def _():
            spread(lambda r0, nr: staged(acc_sh, pa_out, r0, nr, rows_v))
            spread(lambda r0, nr: staged(deg_sh, d_out, r0, nr, z16_v))

        @pl.when(c == 1)
        def _():
            spread(lambda r0, nr: staged(acc_sh, pb_out, r0, nr, rows_v))

    return pl.kernel(body, out_type=out_type, mesh=mesh,
                     scratch_types=scratch,
                     compiler_params=pltpu.CompilerParams(
                         use_tc_tiling_on_sc=False))


def _tc_first(n, f, h, bn):
    grid = (n // bn,)
    fh = h // 2

    def body(x_r, wn_r, wr_r, b_r, ya_r, yb_r, r_r):
        xb = x_r[...]
        y = jnp.dot(xb, wn_r[...], preferred_element_type=F32)
        ya_r[...] = y[:, :fh]
        yb_r[...] = y[:, fh:]
        r_r[...] = jnp.dot(xb, wr_r[...], preferred_element_type=F32) + b_r[...]

    return pl.pallas_call(
        body,
        grid=grid,
        in_specs=[pl.BlockSpec((bn, f), lambda i: (i, 0)),
                  pl.BlockSpec((f, h), lambda i: (0, 0)),
                  pl.BlockSpec((f, h), lambda i: (0, 0)),
                  pl.BlockSpec((1, h), lambda i: (0, 0))],
        out_specs=[pl.BlockSpec((bn, fh), lambda i: (i, 0)),
                   pl.BlockSpec((bn, fh), lambda i: (i, 0)),
                   pl.BlockSpec((bn, h), lambda i: (i, 0))],
        out_shape=[jax.ShapeDtypeStruct((n, fh), F32),
                   jax.ShapeDtypeStruct((n, fh), F32),
                   jax.ShapeDtypeStruct((n, h), F32)],
    )


def _tc_mid(n, h, bn):
    grid = (n // bn,)
    fh = h // 2

    def body(pa_r, pb_r, d_r, r_r, wn_r, wr_r, b_r, ya_r, yb_r, rn_r):
        inv = 1.0 / jnp.maximum(d_r[...][:, 0:1], 1.0)
        agg = jnp.concatenate([pa_r[...], pb_r[...]], axis=1)
        hb = jnp.maximum(agg * inv + r_r[...], 0.0)
        y = jnp.dot(hb, wn_r[...], preferred_element_type=F32)
        ya_r[...] = y[:, :fh]
        yb_r[...] = y[:, fh:]
        rn_r[...] = jnp.dot(hb, wr_r[...], preferred_element_type=F32) + b_r[...]

    return pl.pallas_call(
        body,
        grid=grid,
        in_specs=[pl.BlockSpec((bn, fh), lambda i: (i, 0)),
                  pl.BlockSpec((bn, fh), lambda i: (i, 0)),
                  pl.BlockSpec((bn, DW), lambda i: (i, 0)),
                  pl.BlockSpec((bn, h), lambda i: (i, 0)),
                  pl.BlockSpec((h, h), lambda i: (0, 0)),
                  pl.BlockSpec((h, h), lambda i: (0, 0)),
                  pl.BlockSpec((1, h), lambda i: (0, 0))],
        out_specs=[pl.BlockSpec((bn, fh), lambda i: (i, 0)),
                   pl.BlockSpec((bn, fh), lambda i: (i, 0)),
                   pl.BlockSpec((bn, h), lambda i: (i, 0))],
        out_shape=[jax.ShapeDtypeStruct((n, fh), F32),
                   jax.ShapeDtypeStruct((n, fh), F32),
                   jax.ShapeDtypeStruct((n, h), F32)],
    )


def _tc_final(n, h, g, out, bn):
    grid = (n // bn,)
    fh = h // 2

    def body(pa_r, pb_r, d_r, r_r, bt_r, wfc_r, bfc_r, o_r, accp, accc):
        i = pl.program_id(0)

        @pl.when(i == 0)
        def _():
            accp[...] = jnp.zeros_like(accp)
            accc[...] = jnp.zeros_like(accc)

        inv = 1.0 / jnp.maximum(d_r[...][:, 0:1], 1.0)
        agg = jnp.concatenate([pa_r[...], pb_r[...]], axis=1)
        hb = jnp.maximum(agg * inv + r_r[...], 0.0)
        ids = bt_r[...]
        oh = (ids == lax.broadcasted_iota(jnp.int32, (bn, g), 1)).astype(F32)
        accp[...] += lax.dot_general(oh, hb, (((0,), (0,)), ((), ())),
                                     preferred_element_type=F32)
        accc[...] += jnp.sum(oh, axis=0)[:, None]

        @pl.when(i == pl.num_programs(0) - 1)
        def _():
            pooled = accp[...] / jnp.maximum(accc[...], 1.0)
            o_r[...] = jnp.dot(pooled, wfc_r[...],
                               preferred_element_type=F32) + bfc_r[...]

    return pl.pallas_call(
        body,
        grid=grid,
        in_specs=[pl.BlockSpec((bn, fh), lambda i: (i, 0)),
                  pl.BlockSpec((bn, fh), lambda i: (i, 0)),
                  pl.BlockSpec((bn, DW), lambda i: (i, 0)),
                  pl.BlockSpec((bn, h), lambda i: (i, 0)),
                  pl.BlockSpec((bn, 1), lambda i: (i, 0)),
                  pl.BlockSpec((h, out), lambda i: (0, 0)),
                  pl.BlockSpec((1, out), lambda i: (0, 0))],
        out_specs=pl.BlockSpec((g, out), lambda i: (0, 0)),
        out_shape=jax.ShapeDtypeStruct((g, out), F32),
        scratch_shapes=[pltpu.VMEM((g, h), F32), pltpu.VMEM((g, h), F32)],
    )


@functools.cache
def _build(n, e, fin, h, out, g):
    bn = 2000
    sc = _build_sc_agg(n, e, h)
    tc1 = _tc_first(n, fin, h, bn)
    tcm = _tc_mid(n, h, bn)
    tcf = _tc_final(n, h, g, out, bn)

    def run(x, edge_index, batch,
            wn1, wr1, b1, wn2, wr2, b2, wn3, wr3, b3, wfc, bfc):
        src = edge_index[0].astype(jnp.int32)
        dst = edge_index[1].astype(jnp.int32)
        z = jnp.zeros((n, h // 2), F32)
        z16 = jnp.zeros((n, DW), F32)
        ones = jnp.ones((CH, DW), F32)
        bt = batch.astype(jnp.int32)[:, None]

        ya, yb, r = tc1(x, wn1, wr1, b1[None, :])
        pa, pb, d = sc(ya, yb, src, dst, z, z16, ones)
        ya, yb, r = tcm(pa, pb, d, r, wn2, wr2, b2[None, :])
        pa, pb, _ = sc(ya, yb, src, dst, z, z16, ones)
        ya, yb, r = tcm(pa, pb, d, r, wn3, wr3, b3[None, :])
        pa, pb, _ = sc(ya, yb, src, dst, z, z16, ones)
        return tcf(pa, pb, d, r, bt, wfc, bfc[None, :])

    return run


def kernel(x, edge_index, edge_attr, batch,
           W_neigh1, W_root1, b1, W_neigh2, W_root2, b2,
           W_neigh3, W_root3, b3, W_fc, b_fc):
    n, fin = x.shape
    e = edge_index.shape[1]
    h = W_neigh1.shape[1]
    out = W_fc.shape[1]
    g = 128
    run = _build(n, e, fin, h, out, g)
    return run(x, edge_index, batch,
               W_neigh1, W_root1, b1, W_neigh2, W_root2, b2,
               W_neigh3, W_root3, b3, W_fc, b_fc)


# speedup vs baseline: 7.0806x; 1.8506x over previous
import functools

import jax
import jax.numpy as jnp
from jax import lax
from jax.experimental import pallas as pl
from jax.experimental.pallas import tpu as pltpu
from jax.experimental.pallas import tpu_sc as plsc

F32 = jnp.float32
CH = 128
DW = 16


def _build_sc_agg(n, e, f, with_deg):
    ns = 16
    fh = f // 2
    K = 4
    nsuper = e // (CH * K)
    nb = nsuper // ns
    tail = nsuper - nb * ns
    rpt = (n // ns) // 8 * 8
    rem = n - rpt * ns

    out_type = [jax.ShapeDtypeStruct((n, fh), F32),
                jax.ShapeDtypeStruct((n, fh), F32)]
    scratch = [pltpu.VMEM_SHARED((n, fh), F32),
               pltpu.VMEM((K, CH), jnp.int32),
               pltpu.VMEM((K, CH), jnp.int32),
               pltpu.VMEM((K, CH, fh), F32),
               pltpu.SemaphoreType.DMA,
               pltpu.SemaphoreType.DMA]
    if with_deg:
        out_type += [jax.ShapeDtypeStruct((n, DW), F32)]
        scratch += [pltpu.VMEM_SHARED((n, DW), F32),
                    pltpu.VMEM((CH, DW), F32),
                    pltpu.VMEM((CH, DW), F32)]
    mesh = plsc.VectorSubcoreMesh(core_axis_name="c", subcore_axis_name="s")

    def body(ya_hbm, yb_hbm, src_hbm, dst_hbm, z_hbm, *rest):
        if with_deg:
            (z16_hbm, ones_hbm, pa_out, pb_out, d_out,
             acc_sh, src_v, dst_v, rows_v, sem_g, sem_s,
             deg_sh, ones_v, z16_v) = rest
        else:
            (pa_out, pb_out,
             acc_sh, src_v, dst_v, rows_v, sem_g, sem_s) = rest
        c = lax.axis_index("c")
        s = lax.axis_index("s")

        def staged(src_ref, dst_ref, row0, nrows, stage):
            off = 0
            while off < nrows:
                step = min(CH, nrows - off)
                r = pl.ds(pl.multiple_of(row0 + off, 8), step)
                pltpu.sync_copy(src_ref.at[r], stage.at[pl.ds(0, step)])
                pltpu.sync_copy(stage.at[pl.ds(0, step)], dst_ref.at[r])
                off += step

        def spread(fn):
            fn(pl.multiple_of(s * rpt, 8), rpt)
            if rem:
                @pl.when(s == ns - 1)
                def _():
                    fn(rpt * ns, rem)

        spread(lambda r0, nr: staged(z_hbm, acc_sh, r0, nr, rows_v.at[0]))
        if with_deg:
            @pl.when(c == 0)
            def _():
                spread(lambda r0, nr: staged(z16_hbm, deg_sh, r0, nr, z16_v))
                pltpu.sync_copy(ones_hbm, ones_v)
        plsc.subcore_barrier()

        def do_super(u, tbl, dodeg):
            r = pl.ds(pl.multiple_of(u * K, K), K)
            pltpu.sync_copy(src_hbm.at[r], src_v)
            pltpu.sync_copy(dst_hbm.at[r], dst_v)
            gs = [pltpu.async_copy(tbl.at[src_v.at[q]], rows_v.at[q], sem_g)
                  for q in range(K)]
            for cp in gs:
                cp.wait()
            ss = [pltpu.async_copy(rows_v.at[q], acc_sh.at[dst_v.at[q]],
                                   sem_s, add=True) for q in range(K)]
            if dodeg:
                ss += [pltpu.async_copy(ones_v, deg_sh.at[dst_v.at[q]],
                                        sem_s, add=True) for q in range(K)]
            for cp in ss:
                cp.wait()

        def edge_loop(tbl, dodeg):
            def loop_body(i, carry):
                do_super(s + i * ns, tbl, dodeg)
                return carry
            lax.fori_loop(0, nb, loop_body, 0)
            if tail:
                @pl.when(s < tail)
                def _():
                    do_super(nb * ns + s, tbl, dodeg)

        @pl.when(c == 0)
        def _():
            edge_loop(ya_hbm, with_deg)

        @pl.when(c == 1)
        def _():
            edge_loop(yb_hbm, False)

        plsc.subcore_barrier()

        @pl.when(c == 0)
        def _():
            spread(lambda r0, nr: staged(acc_sh, pa_out, r0, nr, rows_v.at[0]))
            if with_deg:
                spread(lambda r0, nr: staged(deg_sh, d_out, r0, nr, z16_v))

        @pl.when(c == 1)
        def _():
            spread(lambda r0, nr: staged(acc_sh, pb_out, r0, nr, rows_v.at[0]))

    return pl.kernel(body, out_type=tuple(out_type), mesh=mesh,
                     scratch_types=scratch,
                     compiler_params=pltpu.CompilerParams(
                         use_tc_tiling_on_sc=False))


def _tc_first(n, f, h, bn):
    grid = (n // bn,)
    fh = h // 2

    def body(x_r, wn_r, wr_r, b_r, ya_r, yb_r, r_r):
        xb = x_r[...]
        y = jnp.dot(xb, wn_r[...], preferred_element_type=F32)
        ya_r[...] = y[:, :fh]
        yb_r[...] = y[:, fh:]
        r_r[...] = jnp.dot(xb, wr_r[...], preferred_element_type=F32) + b_r[...]

    return pl.pallas_call(
        body,
        grid=grid,
        in_specs=[pl.BlockSpec((bn, f), lambda i: (i, 0)),
                  pl.BlockSpec((f, h), lambda i: (0, 0)),
                  pl.BlockSpec((f, h), lambda i: (0, 0)),
                  pl.BlockSpec((1, h), lambda i: (0, 0))],
        out_specs=[pl.BlockSpec((bn, fh), lambda i: (i, 0)),
                   pl.BlockSpec((bn, fh), lambda i: (i, 0)),
                   pl.BlockSpec((bn, h), lambda i: (i, 0))],
        out_shape=[jax.ShapeDtypeStruct((n, fh), F32),
                   jax.ShapeDtypeStruct((n, fh), F32),
                   jax.ShapeDtypeStruct((n, h), F32)],
    )


def _tc_mid(n, h, bn):
    grid = (n // bn,)
    fh = h // 2

    def body(pa_r, pb_r, d_r, r_r, wn_r, wr_r, b_r, ya_r, yb_r, rn_r):
        inv = 1.0 / jnp.maximum(d_r[...][:, 0:1], 1.0)
        agg = jnp.concatenate([pa_r[...], pb_r[...]], axis=1)
        hb = jnp.maximum(agg * inv + r_r[...], 0.0)
        y = jnp.dot(hb, wn_r[...], preferred_element_type=F32)
        ya_r[...] = y[:, :fh]
        yb_r[...] = y[:, fh:]
        rn_r[...] = jnp.dot(hb, wr_r[...], preferred_element_type=F32) + b_r[...]

    return pl.pallas_call(
        body,
        grid=grid,
        in_specs=[pl.BlockSpec((bn, fh), lambda i: (i, 0)),
                  pl.BlockSpec((bn, fh), lambda i: (i, 0)),
                  pl.BlockSpec((bn, DW), lambda i: (i, 0)),
                  pl.BlockSpec((bn, h), lambda i: (i, 0)),
                  pl.BlockSpec((h, h), lambda i: (0, 0)),
                  pl.BlockSpec((h, h), lambda i: (0, 0)),
                  pl.BlockSpec((1, h), lambda i: (0, 0))],
        out_specs=[pl.BlockSpec((bn, fh), lambda i: (i, 0)),
                   pl.BlockSpec((bn, fh), lambda i: (i, 0)),
                   pl.BlockSpec((bn, h), lambda i: (i, 0))],
        out_shape=[jax.ShapeDtypeStruct((n, fh), F32),
                   jax.ShapeDtypeStruct((n, fh), F32),
                   jax.ShapeDtypeStruct((n, h), F32)],
    )


def _tc_final(n, h, g, out, bn):
    grid = (n // bn,)
    fh = h // 2

    def body(pa_r, pb_r, d_r, r_r, bt_r, wfc_r, bfc_r, o_r, accp, accc):
        i = pl.program_id(0)

        @pl.when(i == 0)
        def _():
            accp[...] = jnp.zeros_like(accp)
            accc[...] = jnp.zeros_like(accc)

        inv = 1.0 / jnp.maximum(d_r[...][:, 0:1], 1.0)
        agg = jnp.concatenate([pa_r[...], pb_r[...]], axis=1)
        hb = jnp.maximum(agg * inv + r_r[...], 0.0)
        ids = bt_r[...]
        oh = (ids == lax.broadcasted_iota(jnp.int32, (bn, g), 1)).astype(F32)
        accp[...] += lax.dot_general(oh, hb, (((0,), (0,)), ((), ())),
                                     preferred_element_type=F32)
        accc[...] += jnp.sum(oh, axis=0)[:, None]

        @pl.when(i == pl.num_programs(0) - 1)
        def _():
            pooled = accp[...] / jnp.maximum(accc[...], 1.0)
            o_r[...] = jnp.dot(pooled, wfc_r[...],
                               preferred_element_type=F32) + bfc_r[...]

    return pl.pallas_call(
        body,
        grid=grid,
        in_specs=[pl.BlockSpec((bn, fh), lambda i: (i, 0)),
                  pl.BlockSpec((bn, fh), lambda i: (i, 0)),
                  pl.BlockSpec((bn, DW), lambda i: (i, 0)),
                  pl.BlockSpec((bn, h), lambda i: (i, 0)),
                  pl.BlockSpec((bn, 1), lambda i: (i, 0)),
                  pl.BlockSpec((h, out), lambda i: (0, 0)),
                  pl.BlockSpec((1, out), lambda i: (0, 0))],
        out_specs=pl.BlockSpec((g, out), lambda i: (0, 0)),
        out_shape=jax.ShapeDtypeStruct((g, out), F32),
        scratch_shapes=[pltpu.VMEM((g, h), F32), pltpu.VMEM((g, h), F32)],
    )


@functools.cache
def _build(n, e, fin, h, out, g):
    bn = 2000
    sc1 = _build_sc_agg(n, e, h, with_deg=True)
    sc23 = _build_sc_agg(n, e, h, with_deg=False)
    tc1 = _tc_first(n, fin, h, bn)
    tcm = _tc_mid(n, h, bn)
    tcf = _tc_final(n, h, g, out, bn)

    def run(x, edge_index, batch,
            wn1, wr1, b1, wn2, wr2, b2, wn3, wr3, b3, wfc, bfc):
        src = edge_index[0].astype(jnp.int32).reshape(-1, CH)
        dst = edge_index[1].astype(jnp.int32).reshape(-1, CH)
        z = jnp.zeros((n, h // 2), F32)
        z16 = jnp.zeros((n, DW), F32)
        ones = jnp.ones((CH, DW), F32)
        bt = batch.astype(jnp.int32)[:, None]

        ya, yb, r = tc1(x, wn1, wr1, b1[None, :])
        pa, pb, d = sc1(ya, yb, src, dst, z, z16, ones)
        ya, yb, r = tcm(pa, pb, d, r, wn2, wr2, b2[None, :])
        pa, pb = sc23(ya, yb, src, dst, z)
        ya, yb, r = tcm(pa, pb, d, r, wn3, wr3, b3[None, :])
        pa, pb = sc23(ya, yb, src, dst, z)
        return tcf(pa, pb, d, r, bt, wfc, bfc[None, :])

    return run


def kernel(x, edge_index, edge_attr, batch,
           W_neigh1, W_root1, b1, W_neigh2, W_root2, b2,
           W_neigh3, W_root3, b3, W_fc, b_fc):
    n, fin = x.shape
    e = edge_index.shape[1]
    h = W_neigh1.shape[1]
    out = W_fc.shape[1]
    g = 128
    run = _build(n, e, fin, h, out, g)
    return run(x, edge_index, batch,
               W_neigh1, W_root1, b1, W_neigh2, W_root2, b2,
               W_neigh3, W_root3, b3, W_fc, b_fc)


# speedup vs baseline: 8.6938x; 1.2278x over previous
import functools

import jax
import jax.numpy as jnp
from jax import lax
from jax.experimental import pallas as pl
from jax.experimental.pallas import tpu as pltpu
from jax.experimental.pallas import tpu_sc as plsc

F32 = jnp.float32
CH = 128
DW = 16


def _build_sc_agg(n, e, f, with_deg):
    ns = 16
    fh = f // 2
    K = 4
    nsuper = e // (CH * K)
    nb = nsuper // ns
    tail = nsuper - nb * ns
    rpt = (n // ns) // 8 * 8
    rem = n - rpt * ns

    out_type = [jax.ShapeDtypeStruct((n, fh), F32),
                jax.ShapeDtypeStruct((n, fh), F32)]
    scratch = [pltpu.VMEM_SHARED((n, fh), F32),
               pltpu.VMEM((2, K, CH), jnp.int32),
               pltpu.VMEM((2, K, CH), jnp.int32),
               pltpu.VMEM((2, K, CH, fh), F32),
               pltpu.SemaphoreType.DMA,
               pltpu.SemaphoreType.DMA,
               pltpu.SemaphoreType.DMA,
               pltpu.SemaphoreType.DMA]
    if with_deg:
        out_type += [jax.ShapeDtypeStruct((n, DW), F32)]
        scratch += [pltpu.VMEM_SHARED((n, DW), F32),
                    pltpu.VMEM((CH, DW), F32),
                    pltpu.VMEM((CH, DW), F32)]
    mesh = plsc.VectorSubcoreMesh(core_axis_name="c", subcore_axis_name="s")

    def body(ya_hbm, yb_hbm, src_hbm, dst_hbm, z_hbm, *rest):
        if with_deg:
            (z16_hbm, ones_hbm, pa_out, pb_out, d_out,
             acc_sh, src_v, dst_v, rows_v, sem_g0, sem_g1, sem_s0, sem_s1,
             deg_sh, ones_v, z16_v) = rest
        else:
            (pa_out, pb_out,
             acc_sh, src_v, dst_v, rows_v,
             sem_g0, sem_g1, sem_s0, sem_s1) = rest
        sem_g = (sem_g0, sem_g1)
        sem_s = (sem_s0, sem_s1)
        c = lax.axis_index("c")
        s = lax.axis_index("s")

        def staged(src_ref, dst_ref, row0, nrows, stage):
            off = 0
            while off < nrows:
                step = min(CH, nrows - off)
                r = pl.ds(pl.multiple_of(row0 + off, 8), step)
                pltpu.sync_copy(src_ref.at[r], stage.at[pl.ds(0, step)])
                pltpu.sync_copy(stage.at[pl.ds(0, step)], dst_ref.at[r])
                off += step

        def spread(fn):
            fn(pl.multiple_of(s * rpt, 8), rpt)
            if rem:
                @pl.when(s == ns - 1)
                def _():
                    fn(rpt * ns, rem)

        spread(lambda r0, nr: staged(z_hbm, acc_sh, r0, nr, rows_v.at[0, 0]))
        if with_deg:
            @pl.when(c == 0)
            def _():
                spread(lambda r0, nr: staged(z16_hbm, deg_sh, r0, nr, z16_v))
                pltpu.sync_copy(ones_hbm, ones_v)
        plsc.subcore_barrier()

        def fire(u, t, tbl):
            r = pl.ds(pl.multiple_of(u * K, K), K)
            pltpu.sync_copy(src_hbm.at[r], src_v.at[t])
            pltpu.sync_copy(dst_hbm.at[r], dst_v.at[t])
            return [pltpu.async_copy(tbl.at[src_v.at[t, q]],
                                     rows_v.at[t, q], sem_g[t])
                    for q in range(K)]

        def mid(gs, t, dodeg):
            for cp in gs:
                cp.wait()
            ss = [pltpu.async_copy(rows_v.at[t, q], acc_sh.at[dst_v.at[t, q]],
                                   sem_s[t], add=True) for q in range(K)]
            if dodeg:
                ss += [pltpu.async_copy(ones_v, deg_sh.at[dst_v.at[t, q]],
                                        sem_s[t], add=True) for q in range(K)]
            return ss

        def do_super(u, tbl, dodeg):
            for cp in mid(fire(u, 0, tbl), 0, dodeg):
                cp.wait()

        def edge_loop(tbl, dodeg):
            def loop_body(i, carry):
                ga = fire(s + (2 * i) * ns, 0, tbl)
                gb = fire(s + (2 * i + 1) * ns, 1, tbl)
                sa = mid(ga, 0, dodeg)
                sb = mid(gb, 1, dodeg)
                for cp in sa + sb:
                    cp.wait()
                return carry
            lax.fori_loop(0, nb // 2, loop_body, 0)
            for j in range(nb // 2 * 2, nb):
                do_super(s + j * ns, tbl, dodeg)
            if tail:
                @pl.when(s < tail)
                def _():
                    do_super(nb * ns + s, tbl, dodeg)

        @pl.when(c == 0)
        def _():
            edge_loop(ya_hbm, with_deg)

        @pl.when(c == 1)
        def _():
            edge_loop(yb_hbm, False)

        plsc.subcore_barrier()

        @pl.when(c == 0)
        def _():
            spread(lambda r0, nr: staged(acc_sh, pa_out, r0, nr,
                                         rows_v.at[0, 0]))
            if with_deg:
                spread(lambda r0, nr: staged(deg_sh, d_out, r0, nr, z16_v))

        @pl.when(c == 1)
        def _():
            spread(lambda r0, nr: staged(acc_sh, pb_out, r0, nr,
                                         rows_v.at[0, 0]))

    return pl.kernel(body, out_type=tuple(out_type), mesh=mesh,
                     scratch_types=scratch,
                     compiler_params=pltpu.CompilerParams(
                         use_tc_tiling_on_sc=False))


def _tc_first(n, f, h, bn):
    grid = (n // bn,)
    fh = h // 2

    def body(x_r, wn_r, wr_r, b_r, ya_r, yb_r, r_r):
        xb = x_r[...]
        y = jnp.dot(xb, wn_r[...], preferred_element_type=F32)
        ya_r[...] = y[:, :fh]
        yb_r[...] = y[:, fh:]
        r_r[...] = jnp.dot(xb, wr_r[...], preferred_element_type=F32) + b_r[...]

    return pl.pallas_call(
        body,
        grid=grid,
        in_specs=[pl.BlockSpec((bn, f), lambda i: (i, 0)),
                  pl.BlockSpec((f, h), lambda i: (0, 0)),
                  pl.BlockSpec((f, h), lambda i: (0, 0)),
                  pl.BlockSpec((1, h), lambda i: (0, 0))],
        out_specs=[pl.BlockSpec((bn, fh), lambda i: (i, 0)),
                   pl.BlockSpec((bn, fh), lambda i: (i, 0)),
                   pl.BlockSpec((bn, h), lambda i: (i, 0))],
        out_shape=[jax.ShapeDtypeStruct((n, fh), F32),
                   jax.ShapeDtypeStruct((n, fh), F32),
                   jax.ShapeDtypeStruct((n, h), F32)],
    )


def _tc_mid(n, h, bn):
    grid = (n // bn,)
    fh = h // 2

    def body(pa_r, pb_r, d_r, r_r, wn_r, wr_r, b_r, ya_r, yb_r, rn_r):
        inv = 1.0 / jnp.maximum(d_r[...][:, 0:1], 1.0)
        agg = jnp.concatenate([pa_r[...], pb_r[...]], axis=1)
        hb = jnp.maximum(agg * inv + r_r[...], 0.0)
        y = jnp.dot(hb, wn_r[...], preferred_element_type=F32)
        ya_r[...] = y[:, :fh]
        yb_r[...] = y[:, fh:]
        rn_r[...] = jnp.dot(hb, wr_r[...], preferred_element_type=F32) + b_r[...]

    return pl.pallas_call(
        body,
        grid=grid,
        in_specs=[pl.BlockSpec((bn, fh), lambda i: (i, 0)),
                  pl.BlockSpec((bn, fh), lambda i: (i, 0)),
                  pl.BlockSpec((bn, DW), lambda i: (i, 0)),
                  pl.BlockSpec((bn, h), lambda i: (i, 0)),
                  pl.BlockSpec((h, h), lambda i: (0, 0)),
                  pl.BlockSpec((h, h), lambda i: (0, 0)),
                  pl.BlockSpec((1, h), lambda i: (0, 0))],
        out_specs=[pl.BlockSpec((bn, fh), lambda i: (i, 0)),
                   pl.BlockSpec((bn, fh), lambda i: (i, 0)),
                   pl.BlockSpec((bn, h), lambda i: (i, 0))],
        out_shape=[jax.ShapeDtypeStruct((n, fh), F32),
                   jax.ShapeDtypeStruct((n, fh), F32),
                   jax.ShapeDtypeStruct((n, h), F32)],
    )


def _tc_final(n, h, g, out, bn):
    grid = (n // bn,)
    fh = h // 2

    def body(pa_r, pb_r, d_r, r_r, bt_r, wfc_r, bfc_r, o_r, accp, accc):
        i = pl.program_id(0)

        @pl.when(i == 0)
        def _():
            accp[...] = jnp.zeros_like(accp)
            accc[...] = jnp.zeros_like(accc)

        inv = 1.0 / jnp.maximum(d_r[...][:, 0:1], 1.0)
        agg = jnp.concatenate([pa_r[...], pb_r[...]], axis=1)
        hb = jnp.maximum(agg * inv + r_r[...], 0.0)
        ids = bt_r[...]
        oh = (ids == lax.broadcasted_iota(jnp.int32, (bn, g), 1)).astype(F32)
        accp[...] += lax.dot_general(oh, hb, (((0,), (0,)), ((), ())),
                                     preferred_element_type=F32)
        accc[...] += jnp.sum(oh, axis=0)[:, None]

        @pl.when(i == pl.num_programs(0) - 1)
        def _():
            pooled = accp[...] / jnp.maximum(accc[...], 1.0)
            o_r[...] = jnp.dot(pooled, wfc_r[...],
                               preferred_element_type=F32) + bfc_r[...]

    return pl.pallas_call(
        body,
        grid=grid,
        in_specs=[pl.BlockSpec((bn, fh), lambda i: (i, 0)),
                  pl.BlockSpec((bn, fh), lambda i: (i, 0)),
                  pl.BlockSpec((bn, DW), lambda i: (i, 0)),
                  pl.BlockSpec((bn, h), lambda i: (i, 0)),
                  pl.BlockSpec((bn, 1), lambda i: (i, 0)),
                  pl.BlockSpec((h, out), lambda i: (0, 0)),
                  pl.BlockSpec((1, out), lambda i: (0, 0))],
        out_specs=pl.BlockSpec((g, out), lambda i: (0, 0)),
        out_shape=jax.ShapeDtypeStruct((g, out), F32),
        scratch_shapes=[pltpu.VMEM((g, h), F32), pltpu.VMEM((g, h), F32)],
    )


@functools.cache
def _build(n, e, fin, h, out, g):
    bn = 2000
    sc1 = _build_sc_agg(n, e, h, with_deg=True)
    sc23 = _build_sc_agg(n, e, h, with_deg=False)
    tc1 = _tc_first(n, fin, h, bn)
    tcm = _tc_mid(n, h, bn)
    tcf = _tc_final(n, h, g, out, bn)

    def run(x, edge_index, batch,
            wn1, wr1, b1, wn2, wr2, b2, wn3, wr3, b3, wfc, bfc):
        src = edge_index[0].astype(jnp.int32).reshape(-1, CH)
        dst = edge_index[1].astype(jnp.int32).reshape(-1, CH)
        z = jnp.zeros((n, h // 2), F32)
        z16 = jnp.zeros((n, DW), F32)
        ones = jnp.ones((CH, DW), F32)
        bt = batch.astype(jnp.int32)[:, None]

        ya, yb, r = tc1(x, wn1, wr1, b1[None, :])
        pa, pb, d = sc1(ya, yb, src, dst, z, z16, ones)
        ya, yb, r = tcm(pa, pb, d, r, wn2, wr2, b2[None, :])
        pa, pb = sc23(ya, yb, src, dst, z)
        ya, yb, r = tcm(pa, pb, d, r, wn3, wr3, b3[None, :])
        pa, pb = sc23(ya, yb, src, dst, z)
        return tcf(pa, pb, d, r, bt, wfc, bfc[None, :])

    return run


def kernel(x, edge_index, edge_attr, batch,
           W_neigh1, W_root1, b1, W_neigh2, W_root2, b2,
           W_neigh3, W_root3, b3, W_fc, b_fc):
    n, fin = x.shape
    e = edge_index.shape[1]
    h = W_neigh1.shape[1]
    out = W_fc.shape[1]
    g = 128
    run = _build(n, e, fin, h, out, g)
    return run(x, edge_index, batch,
               W_neigh1, W_root1, b1, W_neigh2, W_root2, b2,
               W_neigh3, W_root3, b3, W_fc, b_fc)


# speedup vs baseline: 9.6083x; 1.1052x over previous
import functools

import jax
import jax.numpy as jnp
from jax import lax
from jax.experimental import pallas as pl
from jax.experimental.pallas import tpu as pltpu
from jax.experimental.pallas import tpu_sc as plsc

F32 = jnp.float32
CH = 128
DW = 16


def _build_sc_agg(n, e, f, with_deg):
    ns = 16
    fh = f // 2
    K = 4
    nsuper = e // (CH * K)
    nb = nsuper // ns
    tail = nsuper - nb * ns
    rpt = (n // ns) // 8 * 8
    rem = n - rpt * ns
    assert nb >= 3 and nb % 2 == 1, "rolling pipeline assumes odd nb >= 3"

    nbuf = 2
    out_type = [jax.ShapeDtypeStruct((n, fh), F32),
                jax.ShapeDtypeStruct((n, fh), F32)]
    scratch = [pltpu.VMEM_SHARED((n, fh), F32),
               pltpu.VMEM((nbuf, K, CH), jnp.int32),
               pltpu.VMEM((nbuf, K, CH), jnp.int32),
               pltpu.VMEM((nbuf, K, CH, fh), F32)] + \
              [pltpu.SemaphoreType.DMA] * (2 * nbuf)
    if with_deg:
        out_type += [jax.ShapeDtypeStruct((n, DW), F32)]
        scratch += [pltpu.VMEM_SHARED((n, DW), F32),
                    pltpu.VMEM((CH, DW), F32),
                    pltpu.VMEM((CH, DW), F32)]
    mesh = plsc.VectorSubcoreMesh(core_axis_name="c", subcore_axis_name="s")

    def body(ya_hbm, yb_hbm, src_hbm, dst_hbm, z_hbm, *rest):
        if with_deg:
            (z16_hbm, ones_hbm, pa_out, pb_out, d_out,
             acc_sh, src_v, dst_v, rows_v, *sems) = rest
            sems, (deg_sh, ones_v, z16_v) = sems[:4], sems[4:]
        else:
            (pa_out, pb_out, acc_sh, src_v, dst_v, rows_v, *sems) = rest
        sem_g = sems[:2]
        sem_s = sems[2:4]
        c = lax.axis_index("c")
        s = lax.axis_index("s")

        def staged(src_ref, dst_ref, row0, nrows, stage):
            off = 0
            while off < nrows:
                step = min(CH, nrows - off)
                r = pl.ds(pl.multiple_of(row0 + off, 8), step)
                pltpu.sync_copy(src_ref.at[r], stage.at[pl.ds(0, step)])
                pltpu.sync_copy(stage.at[pl.ds(0, step)], dst_ref.at[r])
                off += step

        def spread(fn):
            fn(pl.multiple_of(s * rpt, 8), rpt)
            if rem:
                @pl.when(s == ns - 1)
                def _():
                    fn(rpt * ns, rem)

        spread(lambda r0, nr: staged(z_hbm, acc_sh, r0, nr, rows_v.at[0, 0]))
        if with_deg:
            @pl.when(c == 0)
            def _():
                spread(lambda r0, nr: staged(z16_hbm, deg_sh, r0, nr, z16_v))
                pltpu.sync_copy(ones_hbm, ones_v)

        plsc.subcore_barrier()

        def fire_g(u, t, tbl):
            r = pl.ds(pl.multiple_of(u * K, K), K)
            pltpu.sync_copy(src_hbm.at[r], src_v.at[t])
            pltpu.sync_copy(dst_hbm.at[r], dst_v.at[t])
            for q in range(K):
                pltpu.async_copy(tbl.at[src_v.at[t, q]], rows_v.at[t, q],
                                 sem_g[t])

        def fire_s(t, dodeg):
            for q in range(K):
                pltpu.async_copy(rows_v.at[t, q], acc_sh.at[dst_v.at[t, q]],
                                 sem_s[t], add=True)
            if dodeg:
                for q in range(K):
                    pltpu.async_copy(ones_v, deg_sh.at[dst_v.at[t, q]],
                                     sem_s[t], add=True)

        def drain_g(t):
            for q in range(K):
                pltpu.make_async_copy(z_hbm.at[pl.ds(0, CH)],
                                      rows_v.at[t, q], sem_g[t]).wait()

        def drain_s(t, dodeg):
            for q in range(K):
                pltpu.make_async_copy(z_hbm.at[pl.ds(0, CH)],
                                      rows_v.at[t, q], sem_s[t]).wait()
            if dodeg:
                for q in range(K):
                    pltpu.make_async_copy(z16_hbm.at[pl.ds(0, CH)],
                                          ones_v, sem_s[t]).wait()

        def do_super(u, tbl, dodeg):
            fire_g(u, 0, tbl)
            drain_g(0)
            fire_s(0, dodeg)
            drain_s(0, dodeg)

        def edge_loop(tbl, dodeg):
            npair = nb // 2
            fire_g(s, 0, tbl)

            def loop_body(i, carry):
                fire_g(s + (2 * i + 1) * ns, 1, tbl)
                drain_g(0)
                fire_s(0, dodeg)
                drain_g(1)
                fire_s(1, dodeg)
                drain_s(0, dodeg)
                fire_g(s + (2 * i + 2) * ns, 0, tbl)
                drain_s(1, dodeg)
                return carry
            lax.fori_loop(0, npair, loop_body, 0)
            drain_g(0)
            fire_s(0, dodeg)
            drain_s(0, dodeg)
            if tail:
                @pl.when(s < tail)
                def _():
                    do_super(nb * ns + s, tbl, dodeg)

        @pl.when(c == 0)
        def _():
            edge_loop(ya_hbm, with_deg)

        @pl.when(c == 1)
        def _():
            edge_loop(yb_hbm, False)

        plsc.subcore_barrier()

        @pl.when(c == 0)
        def _():
            spread(lambda r0, nr: staged(acc_sh, pa_out, r0, nr,
                                         rows_v.at[0, 0]))
            if with_deg:
                spread(lambda r0, nr: staged(deg_sh, d_out, r0, nr, z16_v))

        @pl.when(c == 1)
        def _():
            spread(lambda r0, nr: staged(acc_sh, pb_out, r0, nr,
                                         rows_v.at[0, 0]))

    return pl.kernel(body, out_type=tuple(out_type), mesh=mesh,
                     scratch_types=scratch,
                     compiler_params=pltpu.CompilerParams(
                         use_tc_tiling_on_sc=False))


def _tc_first(n, f, h, bn):
    grid = (n // bn,)
    fh = h // 2

    def body(x_r, wn_r, wr_r, b_r, ya_r, yb_r, r_r):
        xb = x_r[...]
        y = jnp.dot(xb, wn_r[...], preferred_element_type=F32)
        ya_r[...] = y[:, :fh]
        yb_r[...] = y[:, fh:]
        r_r[...] = jnp.dot(xb, wr_r[...], preferred_element_type=F32) + b_r[...]

    return pl.pallas_call(
        body,
        grid=grid,
        in_specs=[pl.BlockSpec((bn, f), lambda i: (i, 0)),
                  pl.BlockSpec((f, h), lambda i: (0, 0)),
                  pl.BlockSpec((f, h), lambda i: (0, 0)),
                  pl.BlockSpec((1, h), lambda i: (0, 0))],
        out_specs=[pl.BlockSpec((bn, fh), lambda i: (i, 0)),
                   pl.BlockSpec((bn, fh), lambda i: (i, 0)),
                   pl.BlockSpec((bn, h), lambda i: (i, 0))],
        out_shape=[jax.ShapeDtypeStruct((n, fh), F32),
                   jax.ShapeDtypeStruct((n, fh), F32),
                   jax.ShapeDtypeStruct((n, h), F32)],
    )


def _tc_mid(n, h, bn):
    grid = (n // bn,)
    fh = h // 2

    def body(pa_r, pb_r, d_r, r_r, wn_r, wr_r, b_r, ya_r, yb_r, rn_r):
        inv = 1.0 / jnp.maximum(d_r[...][:, 0:1], 1.0)
        agg = jnp.concatenate([pa_r[...], pb_r[...]], axis=1)
        hb = jnp.maximum(agg * inv + r_r[...], 0.0)
        y = jnp.dot(hb, wn_r[...], preferred_element_type=F32)
        ya_r[...] = y[:, :fh]
        yb_r[...] = y[:, fh:]
        rn_r[...] = jnp.dot(hb, wr_r[...], preferred_element_type=F32) + b_r[...]

    return pl.pallas_call(
        body,
        grid=grid,
        in_specs=[pl.BlockSpec((bn, fh), lambda i: (i, 0)),
                  pl.BlockSpec((bn, fh), lambda i: (i, 0)),
                  pl.BlockSpec((bn, DW), lambda i: (i, 0)),
                  pl.BlockSpec((bn, h), lambda i: (i, 0)),
                  pl.BlockSpec((h, h), lambda i: (0, 0)),
                  pl.BlockSpec((h, h), lambda i: (0, 0)),
                  pl.BlockSpec((1, h), lambda i: (0, 0))],
        out_specs=[pl.BlockSpec((bn, fh), lambda i: (i, 0)),
                   pl.BlockSpec((bn, fh), lambda i: (i, 0)),
                   pl.BlockSpec((bn, h), lambda i: (i, 0))],
        out_shape=[jax.ShapeDtypeStruct((n, fh), F32),
                   jax.ShapeDtypeStruct((n, fh), F32),
                   jax.ShapeDtypeStruct((n, h), F32)],
    )


def _tc_final(n, h, g, out, bn):
    grid = (n // bn,)
    fh = h // 2

    def body(pa_r, pb_r, d_r, r_r, bt_r, wfc_r, bfc_r, o_r, accp, accc):
        i = pl.program_id(0)

        @pl.when(i == 0)
        def _():
            accp[...] = jnp.zeros_like(accp)
            accc[...] = jnp.zeros_like(accc)

        inv = 1.0 / jnp.maximum(d_r[...][:, 0:1], 1.0)
        agg = jnp.concatenate([pa_r[...], pb_r[...]], axis=1)
        hb = jnp.maximum(agg * inv + r_r[...], 0.0)
        ids = bt_r[...]
        oh = (ids == lax.broadcasted_iota(jnp.int32, (bn, g), 1)).astype(F32)
        accp[...] += lax.dot_general(oh, hb, (((0,), (0,)), ((), ())),
                                     preferred_element_type=F32)
        accc[...] += jnp.sum(oh, axis=0)[:, None]

        @pl.when(i == pl.num_programs(0) - 1)
        def _():
            pooled = accp[...] / jnp.maximum(accc[...], 1.0)
            o_r[...] = jnp.dot(pooled, wfc_r[...],
                               preferred_element_type=F32) + bfc_r[...]

    return pl.pallas_call(
        body,
        grid=grid,
        in_specs=[pl.BlockSpec((bn, fh), lambda i: (i, 0)),
                  pl.BlockSpec((bn, fh), lambda i: (i, 0)),
                  pl.BlockSpec((bn, DW), lambda i: (i, 0)),
                  pl.BlockSpec((bn, h), lambda i: (i, 0)),
                  pl.BlockSpec((bn, 1), lambda i: (i, 0)),
                  pl.BlockSpec((h, out), lambda i: (0, 0)),
                  pl.BlockSpec((1, out), lambda i: (0, 0))],
        out_specs=pl.BlockSpec((g, out), lambda i: (0, 0)),
        out_shape=jax.ShapeDtypeStruct((g, out), F32),
        scratch_shapes=[pltpu.VMEM((g, h), F32), pltpu.VMEM((g, h), F32)],
    )


@functools.cache
def _build(n, e, fin, h, out, g):
    bn = 2000
    sc1 = _build_sc_agg(n, e, h, with_deg=True)
    sc23 = _build_sc_agg(n, e, h, with_deg=False)
    tc1 = _tc_first(n, fin, h, bn)
    tcm = _tc_mid(n, h, bn)
    tcf = _tc_final(n, h, g, out, bn)

    def run(x, edge_index, batch,
            wn1, wr1, b1, wn2, wr2, b2, wn3, wr3, b3, wfc, bfc):
        src = edge_index[0].astype(jnp.int32).reshape(-1, CH)
        dst = edge_index[1].astype(jnp.int32).reshape(-1, CH)
        z = jnp.zeros((n, h // 2), F32)
        z16 = jnp.zeros((n, DW), F32)
        ones = jnp.ones((CH, DW), F32)
        bt = batch.astype(jnp.int32)[:, None]

        ya, yb, r = tc1(x, wn1, wr1, b1[None, :])
        pa, pb, d = sc1(ya, yb, src, dst, z, z16, ones)
        ya, yb, r = tcm(pa, pb, d, r, wn2, wr2, b2[None, :])
        pa, pb = sc23(ya, yb, src, dst, z)
        ya, yb, r = tcm(pa, pb, d, r, wn3, wr3, b3[None, :])
        pa, pb = sc23(ya, yb, src, dst, z)
        return tcf(pa, pb, d, r, bt, wfc, bfc[None, :])

    return run


def kernel(x, edge_index, edge_attr, batch,
           W_neigh1, W_root1, b1, W_neigh2, W_root2, b2,
           W_neigh3, W_root3, b3, W_fc, b_fc):
    n, fin = x.shape
    e = edge_index.shape[1]
    h = W_neigh1.shape[1]
    out = W_fc.shape[1]
    g = 128
    run = _build(n, e, fin, h, out, g)
    return run(x, edge_index, batch,
               W_neigh1, W_root1, b1, W_neigh2, W_root2, b2,
               W_neigh3, W_root3, b3, W_fc, b_fc)


# speedup vs baseline: 10.2595x; 1.0678x over previous
import functools

import jax
import jax.numpy as jnp
from jax import lax
from jax.experimental import pallas as pl
from jax.experimental.pallas import tpu as pltpu
from jax.experimental.pallas import tpu_sc as plsc

F32 = jnp.float32
CH = 128
DW = 16


def _build_sc_agg(n, e, f, with_deg):
    ns = 16
    fh = f // 2
    K = 4
    nsuper = e // (CH * K)
    nb = nsuper // ns
    tail = nsuper - nb * ns
    rpt = (n // ns) // 8 * 8
    rem = n - rpt * ns
    assert nb >= 3 and nb % 2 == 1, "rolling pipeline assumes odd nb >= 3"

    nbuf = 2
    out_type = [jax.ShapeDtypeStruct((n, fh), F32),
                jax.ShapeDtypeStruct((n, fh), F32)]
    scratch = [pltpu.VMEM_SHARED((n, fh), F32),
               pltpu.VMEM((nbuf, K, CH), jnp.int32),
               pltpu.VMEM((nbuf, K, CH), jnp.int32),
               pltpu.VMEM((nbuf, K, CH, fh), F32)] + \
              [pltpu.SemaphoreType.DMA] * (3 * nbuf)
    if with_deg:
        out_type += [jax.ShapeDtypeStruct((n, DW), F32)]
        scratch += [pltpu.VMEM_SHARED((n, DW), F32),
                    pltpu.VMEM((CH, DW), F32),
                    pltpu.VMEM((CH, DW), F32)]
    mesh = plsc.VectorSubcoreMesh(core_axis_name="c", subcore_axis_name="s")

    def body(ya_hbm, yb_hbm, src_hbm, dst_hbm, z_hbm, *rest):
        if with_deg:
            (z16_hbm, ones_hbm, pa_out, pb_out, d_out,
             acc_sh, src_v, dst_v, rows_v, *sems) = rest
            sems, (deg_sh, ones_v, z16_v) = sems[:6], sems[6:]
        else:
            (pa_out, pb_out, acc_sh, src_v, dst_v, rows_v, *sems) = rest
        sem_g = sems[:2]
        sem_s = sems[2:4]
        sem_i = sems[4:6]
        c = lax.axis_index("c")
        s = lax.axis_index("s")

        def staged(src_ref, dst_ref, row0, nrows, stage):
            off = 0
            while off < nrows:
                step = min(CH, nrows - off)
                r = pl.ds(pl.multiple_of(row0 + off, 8), step)
                pltpu.sync_copy(src_ref.at[r], stage.at[pl.ds(0, step)])
                pltpu.sync_copy(stage.at[pl.ds(0, step)], dst_ref.at[r])
                off += step

        def spread(fn):
            fn(pl.multiple_of(s * rpt, 8), rpt)
            if rem:
                @pl.when(s == ns - 1)
                def _():
                    fn(rpt * ns, rem)

        spread(lambda r0, nr: staged(z_hbm, acc_sh, r0, nr, rows_v.at[0, 0]))
        if with_deg:
            @pl.when(c == 0)
            def _():
                spread(lambda r0, nr: staged(z16_hbm, deg_sh, r0, nr, z16_v))
                pltpu.sync_copy(ones_hbm, ones_v)

        plsc.subcore_barrier()

        def fire_i(u, t):
            r = pl.ds(pl.multiple_of(u * K, K), K)
            pltpu.async_copy(src_hbm.at[r], src_v.at[t], sem_i[t])
            pltpu.async_copy(dst_hbm.at[r], dst_v.at[t], sem_i[t])

        def drain_i(t):
            pltpu.make_async_copy(src_hbm.at[pl.ds(0, K)], src_v.at[t],
                                  sem_i[t]).wait()
            pltpu.make_async_copy(dst_hbm.at[pl.ds(0, K)], dst_v.at[t],
                                  sem_i[t]).wait()

        def fire_g(t, tbl):
            for q in range(K):
                pltpu.async_copy(tbl.at[src_v.at[t, q]], rows_v.at[t, q],
                                 sem_g[t])

        def fire_s(t, dodeg):
            for q in range(K):
                pltpu.async_copy(rows_v.at[t, q], acc_sh.at[dst_v.at[t, q]],
                                 sem_s[t], add=True)
            if dodeg:
                for q in range(K):
                    pltpu.async_copy(ones_v, deg_sh.at[dst_v.at[t, q]],
                                     sem_s[t], add=True)

        def drain_g(t):
            for q in range(K):
                pltpu.make_async_copy(z_hbm.at[pl.ds(0, CH)],
                                      rows_v.at[t, q], sem_g[t]).wait()

        def drain_s(t, dodeg):
            for q in range(K):
                pltpu.make_async_copy(z_hbm.at[pl.ds(0, CH)],
                                      rows_v.at[t, q], sem_s[t]).wait()
            if dodeg:
                for q in range(K):
                    pltpu.make_async_copy(z16_hbm.at[pl.ds(0, CH)],
                                          ones_v, sem_s[t]).wait()

        def do_super(u, tbl, dodeg):
            fire_i(u, 0)
            drain_i(0)
            fire_g(0, tbl)
            drain_g(0)
            fire_s(0, dodeg)
            drain_s(0, dodeg)

        def edge_loop(tbl, dodeg):
            npair = nb // 2

            def stage(i, last):
                drain_g(0)
                fire_s(0, dodeg)
                drain_i(1)
                fire_g(1, tbl)
                drain_s(0, dodeg)
                fire_i(s + (2 * i + 2) * ns, 0)
                drain_g(1)
                fire_s(1, dodeg)
                drain_i(0)
                fire_g(0, tbl)
                drain_s(1, dodeg)
                if not last:
                    fire_i(s + (2 * i + 3) * ns, 1)

            fire_i(s, 0)
            drain_i(0)
            fire_g(0, tbl)
            fire_i(s + 1 * ns, 1)

            def loop_body(i, carry):
                stage(i, last=False)
                return carry
            lax.fori_loop(0, npair - 1, loop_body, 0)
            stage(npair - 1, last=True)
            drain_g(0)
            fire_s(0, dodeg)
            drain_s(0, dodeg)
            if tail:
                @pl.when(s < tail)
                def _():
                    do_super(nb * ns + s, tbl, dodeg)

        @pl.when(c == 0)
        def _():
            edge_loop(ya_hbm, with_deg)

        @pl.when(c == 1)
        def _():
            edge_loop(yb_hbm, False)

        plsc.subcore_barrier()

        @pl.when(c == 0)
        def _():
            spread(lambda r0, nr: staged(acc_sh, pa_out, r0, nr,
                                         rows_v.at[0, 0]))
            if with_deg:
                spread(lambda r0, nr: staged(deg_sh, d_out, r0, nr, z16_v))

        @pl.when(c == 1)
        def _():
            spread(lambda r0, nr: staged(acc_sh, pb_out, r0, nr,
                                         rows_v.at[0, 0]))

    return pl.kernel(body, out_type=tuple(out_type), mesh=mesh,
                     scratch_types=scratch,
                     compiler_params=pltpu.CompilerParams(
                         use_tc_tiling_on_sc=False))


def _tc_first(n, f, h, bn):
    grid = (n // bn,)
    fh = h // 2

    def body(x_r, wn_r, wr_r, b_r, ya_r, yb_r, r_r):
        xb = x_r[...]
        y = jnp.dot(xb, wn_r[...], preferred_element_type=F32)
        ya_r[...] = y[:, :fh]
        yb_r[...] = y[:, fh:]
        r_r[...] = jnp.dot(xb, wr_r[...], preferred_element_type=F32) + b_r[...]

    return pl.pallas_call(
        body,
        grid=grid,
        in_specs=[pl.BlockSpec((bn, f), lambda i: (i, 0)),
                  pl.BlockSpec((f, h), lambda i: (0, 0)),
                  pl.BlockSpec((f, h), lambda i: (0, 0)),
                  pl.BlockSpec((1, h), lambda i: (0, 0))],
        out_specs=[pl.BlockSpec((bn, fh), lambda i: (i, 0)),
                   pl.BlockSpec((bn, fh), lambda i: (i, 0)),
                   pl.BlockSpec((bn, h), lambda i: (i, 0))],
        out_shape=[jax.ShapeDtypeStruct((n, fh), F32),
                   jax.ShapeDtypeStruct((n, fh), F32),
                   jax.ShapeDtypeStruct((n, h), F32)],
    )


def _tc_mid(n, h, bn):
    grid = (n // bn,)
    fh = h // 2

    def body(pa_r, pb_r, d_r, r_r, wn_r, wr_r, b_r, ya_r, yb_r, rn_r):
        inv = 1.0 / jnp.maximum(d_r[...][:, 0:1], 1.0)
        agg = jnp.concatenate([pa_r[...], pb_r[...]], axis=1)
        hb = jnp.maximum(agg * inv + r_r[...], 0.0)
        y = jnp.dot(hb, wn_r[...], preferred_element_type=F32)
        ya_r[...] = y[:, :fh]
        yb_r[...] = y[:, fh:]
        rn_r[...] = jnp.dot(hb, wr_r[...], preferred_element_type=F32) + b_r[...]

    return pl.pallas_call(
        body,
        grid=grid,
        in_specs=[pl.BlockSpec((bn, fh), lambda i: (i, 0)),
                  pl.BlockSpec((bn, fh), lambda i: (i, 0)),
                  pl.BlockSpec((bn, DW), lambda i: (i, 0)),
                  pl.BlockSpec((bn, h), lambda i: (i, 0)),
                  pl.BlockSpec((h, h), lambda i: (0, 0)),
                  pl.BlockSpec((h, h), lambda i: (0, 0)),
                  pl.BlockSpec((1, h), lambda i: (0, 0))],
        out_specs=[pl.BlockSpec((bn, fh), lambda i: (i, 0)),
                   pl.BlockSpec((bn, fh), lambda i: (i, 0)),
                   pl.BlockSpec((bn, h), lambda i: (i, 0))],
        out_shape=[jax.ShapeDtypeStruct((n, fh), F32),
                   jax.ShapeDtypeStruct((n, fh), F32),
                   jax.ShapeDtypeStruct((n, h), F32)],
    )


def _tc_final(n, h, g, out, bn):
    grid = (n // bn,)
    fh = h // 2

    def body(pa_r, pb_r, d_r, r_r, bt_r, wfc_r, bfc_r, o_r, accp, accc):
        i = pl.program_id(0)

        @pl.when(i == 0)
        def _():
            accp[...] = jnp.zeros_like(accp)
            accc[...] = jnp.zeros_like(accc)

        inv = 1.0 / jnp.maximum(d_r[...][:, 0:1], 1.0)
        agg = jnp.concatenate([pa_r[...], pb_r[...]], axis=1)
        hb = jnp.maximum(agg * inv + r_r[...], 0.0)
        ids = bt_r[...]
        oh = (ids == lax.broadcasted_iota(jnp.int32, (bn, g), 1)).astype(F32)
        accp[...] += lax.dot_general(oh, hb, (((0,), (0,)), ((), ())),
                                     preferred_element_type=F32)
        accc[...] += jnp.sum(oh, axis=0)[:, None]

        @pl.when(i == pl.num_programs(0) - 1)
        def _():
            pooled = accp[...] / jnp.maximum(accc[...], 1.0)
            o_r[...] = jnp.dot(pooled, wfc_r[...],
                               preferred_element_type=F32) + bfc_r[...]

    return pl.pallas_call(
        body,
        grid=grid,
        in_specs=[pl.BlockSpec((bn, fh), lambda i: (i, 0)),
                  pl.BlockSpec((bn, fh), lambda i: (i, 0)),
                  pl.BlockSpec((bn, DW), lambda i: (i, 0)),
                  pl.BlockSpec((bn, h), lambda i: (i, 0)),
                  pl.BlockSpec((bn, 1), lambda i: (i, 0)),
                  pl.BlockSpec((h, out), lambda i: (0, 0)),
                  pl.BlockSpec((1, out), lambda i: (0, 0))],
        out_specs=pl.BlockSpec((g, out), lambda i: (0, 0)),
        out_shape=jax.ShapeDtypeStruct((g, out), F32),
        scratch_shapes=[pltpu.VMEM((g, h), F32), pltpu.VMEM((g, h), F32)],
    )


@functools.cache
def _build(n, e, fin, h, out, g):
    bn = 2000
    sc1 = _build_sc_agg(n, e, h, with_deg=True)
    sc23 = _build_sc_agg(n, e, h, with_deg=False)
    tc1 = _tc_first(n, fin, h, bn)
    tcm = _tc_mid(n, h, bn)
    tcf = _tc_final(n, h, g, out, bn)

    def run(x, edge_index, batch,
            wn1, wr1, b1, wn2, wr2, b2, wn3, wr3, b3, wfc, bfc):
        src = edge_index[0].astype(jnp.int32).reshape(-1, CH)
        dst = edge_index[1].astype(jnp.int32).reshape(-1, CH)
        z = jnp.zeros((n, h // 2), F32)
        z16 = jnp.zeros((n, DW), F32)
        ones = jnp.ones((CH, DW), F32)
        bt = batch.astype(jnp.int32)[:, None]

        ya, yb, r = tc1(x, wn1, wr1, b1[None, :])
        pa, pb, d = sc1(ya, yb, src, dst, z, z16, ones)
        ya, yb, r = tcm(pa, pb, d, r, wn2, wr2, b2[None, :])
        pa, pb = sc23(ya, yb, src, dst, z)
        ya, yb, r = tcm(pa, pb, d, r, wn3, wr3, b3[None, :])
        pa, pb = sc23(ya, yb, src, dst, z)
        return tcf(pa, pb, d, r, bt, wfc, bfc[None, :])

    return run


def kernel(x, edge_index, edge_attr, batch,
           W_neigh1, W_root1, b1, W_neigh2, W_root2, b2,
           W_neigh3, W_root3, b3, W_fc, b_fc):
    n, fin = x.shape
    e = edge_index.shape[1]
    h = W_neigh1.shape[1]
    out = W_fc.shape[1]
    g = 128
    run = _build(n, e, fin, h, out, g)
    return run(x, edge_index, batch,
               W_neigh1, W_root1, b1, W_neigh2, W_root2, b2,
               W_neigh3, W_root3, b3, W_fc, b_fc)


# speedup vs baseline: 10.6192x; 1.0351x over previous
import functools

import jax
import jax.numpy as jnp
from jax import lax
from jax.experimental import pallas as pl
from jax.experimental.pallas import tpu as pltpu
from jax.experimental.pallas import tpu_sc as plsc

F32 = jnp.float32
CH = 128
DW = 16


def _build_sc_agg(n, e, f, with_deg):
    ns = 16
    fh = f // 2
    K = 4
    nsuper = e // (CH * K)
    nb = nsuper // ns
    tail = nsuper - nb * ns
    rpt = (n // ns) // 8 * 8
    rem = n - rpt * ns
    assert nb >= 3 and nb % 2 == 1, "rolling pipeline assumes odd nb >= 3"

    nbuf = 2
    out_type = [jax.ShapeDtypeStruct((n, fh), F32),
                jax.ShapeDtypeStruct((n, fh), F32)]
    scratch = [pltpu.VMEM_SHARED((n, fh), F32),
               pltpu.VMEM((nbuf, K, CH), jnp.int32),
               pltpu.VMEM((nbuf, K, CH), jnp.int32),
               pltpu.VMEM((nbuf, K, CH, fh), F32)] + \
              [pltpu.SemaphoreType.DMA] * (3 * nbuf)
    if with_deg:
        out_type += [jax.ShapeDtypeStruct((n, DW), F32)]
        scratch += [pltpu.VMEM_SHARED((n, DW), F32),
                    pltpu.VMEM((CH, DW), F32),
                    pltpu.VMEM((CH, DW), F32)]
    mesh = plsc.VectorSubcoreMesh(core_axis_name="c", subcore_axis_name="s")

    def body(ya_hbm, yb_hbm, src_hbm, dst_hbm, z_hbm, *rest):
        if with_deg:
            (z16_hbm, ones_hbm, pa_out, pb_out, d_out,
             acc_sh, src_v, dst_v, rows_v, *sems) = rest
            sems, (deg_sh, ones_v, z16_v) = sems[:6], sems[6:]
        else:
            (pa_out, pb_out, acc_sh, src_v, dst_v, rows_v, *sems) = rest
        sem_g = sems[:2]
        sem_s = sems[2:4]
        sem_i = sems[4:6]
        c = lax.axis_index("c")
        s = lax.axis_index("s")

        def staged(src_ref, dst_ref, row0, nrows, stages, semA, semB):
            chunks, off = [], 0
            while off < nrows:
                step = min(CH, nrows - off)
                chunks.append((pl.ds(pl.multiple_of(row0 + off, 8), step),
                               step, stages[len(chunks) % len(stages)]))
                off += step
            nch, depth = len(chunks), len(stages)
            loads = [pltpu.async_copy(src_ref.at[chunks[i][0]],
                                      chunks[i][2].at[pl.ds(0, chunks[i][1])],
                                      semA)
                     for i in range(min(depth, nch))]
            stores = []
            for i in range(nch):
                r, k, st = chunks[i]
                loads[i].wait()
                stores.append(pltpu.async_copy(st.at[pl.ds(0, k)],
                                               dst_ref.at[r], semB))
                j = i + depth
                if j < nch:
                    stores[i].wait()
                    rj, kj, stj = chunks[j]
                    loads.append(pltpu.async_copy(
                        src_ref.at[rj], stj.at[pl.ds(0, kj)], semA))
            for i in range(max(0, nch - depth), nch):
                stores[i].wait()

        def spread(fn):
            fn(pl.multiple_of(s * rpt, 8), rpt)
            if rem:
                @pl.when(s == ns - 1)
                def _():
                    fn(rpt * ns, rem)

        stage8 = [rows_v.at[t, q] for t in range(nbuf) for q in range(K)]

        spread(lambda r0, nr: staged(z_hbm, acc_sh, r0, nr, stage8,
                                     sem_g[0], sem_s[0]))
        if with_deg:
            @pl.when(c == 0)
            def _():
                spread(lambda r0, nr: staged(z16_hbm, deg_sh, r0, nr,
                                             [ones_v, z16_v],
                                             sem_g[1], sem_s[1]))
                pltpu.sync_copy(ones_hbm, ones_v)

        plsc.subcore_barrier()

        def fire_i(u, t):
            r = pl.ds(pl.multiple_of(u * K, K), K)
            pltpu.async_copy(src_hbm.at[r], src_v.at[t], sem_i[t])
            pltpu.async_copy(dst_hbm.at[r], dst_v.at[t], sem_i[t])

        def drain_i(t):
            pltpu.make_async_copy(src_hbm.at[pl.ds(0, K)], src_v.at[t],
                                  sem_i[t]).wait()
            pltpu.make_async_copy(dst_hbm.at[pl.ds(0, K)], dst_v.at[t],
                                  sem_i[t]).wait()

        def fire_g(t, tbl):
            for q in range(K):
                pltpu.async_copy(tbl.at[src_v.at[t, q]], rows_v.at[t, q],
                                 sem_g[t])

        def fire_s(t, dodeg):
            for q in range(K):
                pltpu.async_copy(rows_v.at[t, q], acc_sh.at[dst_v.at[t, q]],
                                 sem_s[t], add=True)
            if dodeg:
                for q in range(K):
                    pltpu.async_copy(ones_v, deg_sh.at[dst_v.at[t, q]],
                                     sem_s[t], add=True)

        def drain_g(t):
            for q in range(K):
                pltpu.make_async_copy(z_hbm.at[pl.ds(0, CH)],
                                      rows_v.at[t, q], sem_g[t]).wait()

        def drain_s(t, dodeg):
            for q in range(K):
                pltpu.make_async_copy(z_hbm.at[pl.ds(0, CH)],
                                      rows_v.at[t, q], sem_s[t]).wait()
            if dodeg:
                for q in range(K):
                    pltpu.make_async_copy(z16_hbm.at[pl.ds(0, CH)],
                                          ones_v, sem_s[t]).wait()

        def do_super(u, tbl, dodeg):
            fire_i(u, 0)
            drain_i(0)
            fire_g(0, tbl)
            drain_g(0)
            fire_s(0, dodeg)
            drain_s(0, dodeg)

        def edge_loop(tbl, dodeg):
            npair = nb // 2

            def stage(i, last):
                drain_g(0)
                fire_s(0, dodeg)
                drain_i(1)
                fire_g(1, tbl)
                drain_s(0, dodeg)
                fire_i(s + (2 * i + 2) * ns, 0)
                drain_g(1)
                fire_s(1, dodeg)
                drain_i(0)
                fire_g(0, tbl)
                drain_s(1, dodeg)
                if not last:
                    fire_i(s + (2 * i + 3) * ns, 1)

            fire_i(s, 0)
            drain_i(0)
            fire_g(0, tbl)
            fire_i(s + 1 * ns, 1)

            def loop_body(i, carry):
                stage(i, last=False)
                return carry
            lax.fori_loop(0, npair - 1, loop_body, 0)
            stage(npair - 1, last=True)
            drain_g(0)
            fire_s(0, dodeg)
            drain_s(0, dodeg)
            if tail:
                @pl.when(s < tail)
                def _():
                    do_super(nb * ns + s, tbl, dodeg)

        @pl.when(c == 0)
        def _():
            edge_loop(ya_hbm, with_deg)

        @pl.when(c == 1)
        def _():
            edge_loop(yb_hbm, False)

        plsc.subcore_barrier()

        @pl.when(c == 0)
        def _():
            spread(lambda r0, nr: staged(acc_sh, pa_out, r0, nr, stage8,
                                         sem_g[0], sem_s[0]))
            if with_deg:
                spread(lambda r0, nr: staged(deg_sh, d_out, r0, nr,
                                             [ones_v, z16_v],
                                             sem_g[1], sem_s[1]))

        @pl.when(c == 1)
        def _():
            spread(lambda r0, nr: staged(acc_sh, pb_out, r0, nr, stage8,
                                         sem_g[0], sem_s[0]))

    return pl.kernel(body, out_type=tuple(out_type), mesh=mesh,
                     scratch_types=scratch,
                     compiler_params=pltpu.CompilerParams(
                         use_tc_tiling_on_sc=False))


def _tc_first(n, f, h, bn):
    grid = (n // bn,)
    fh = h // 2

    def body(x_r, wn_r, wr_r, b_r, ya_r, yb_r, r_r):
        xb = x_r[...]
        y = jnp.dot(xb, wn_r[...], preferred_element_type=F32)
        ya_r[...] = y[:, :fh]
        yb_r[...] = y[:, fh:]
        r_r[...] = jnp.dot(xb, wr_r[...], preferred_element_type=F32) + b_r[...]

    return pl.pallas_call(
        body,
        grid=grid,
        in_specs=[pl.BlockSpec((bn, f), lambda i: (i, 0)),
                  pl.BlockSpec((f, h), lambda i: (0, 0)),
                  pl.BlockSpec((f, h), lambda i: (0, 0)),
                  pl.BlockSpec((1, h), lambda i: (0, 0))],
        out_specs=[pl.BlockSpec((bn, fh), lambda i: (i, 0)),
                   pl.BlockSpec((bn, fh), lambda i: (i, 0)),
                   pl.BlockSpec((bn, h), lambda i: (i, 0))],
        out_shape=[jax.ShapeDtypeStruct((n, fh), F32),
                   jax.ShapeDtypeStruct((n, fh), F32),
                   jax.ShapeDtypeStruct((n, h), F32)],
    )


def _tc_mid(n, h, bn):
    grid = (n // bn,)
    fh = h // 2

    def body(pa_r, pb_r, d_r, r_r, wn_r, wr_r, b_r, ya_r, yb_r, rn_r):
        inv = 1.0 / jnp.maximum(d_r[...][:, 0:1], 1.0)
        agg = jnp.concatenate([pa_r[...], pb_r[...]], axis=1)
        hb = jnp.maximum(agg * inv + r_r[...], 0.0)
        y = jnp.dot(hb, wn_r[...], preferred_element_type=F32)
        ya_r[...] = y[:, :fh]
        yb_r[...] = y[:, fh:]
        rn_r[...] = jnp.dot(hb, wr_r[...], preferred_element_type=F32) + b_r[...]

    return pl.pallas_call(
        body,
        grid=grid,
        in_specs=[pl.BlockSpec((bn, fh), lambda i: (i, 0)),
                  pl.BlockSpec((bn, fh), lambda i: (i, 0)),
                  pl.BlockSpec((bn, DW), lambda i: (i, 0)),
                  pl.BlockSpec((bn, h), lambda i: (i, 0)),
                  pl.BlockSpec((h, h), lambda i: (0, 0)),
                  pl.BlockSpec((h, h), lambda i: (0, 0)),
                  pl.BlockSpec((1, h), lambda i: (0, 0))],
        out_specs=[pl.BlockSpec((bn, fh), lambda i: (i, 0)),
                   pl.BlockSpec((bn, fh), lambda i: (i, 0)),
                   pl.BlockSpec((bn, h), lambda i: (i, 0))],
        out_shape=[jax.ShapeDtypeStruct((n, fh), F32),
                   jax.ShapeDtypeStruct((n, fh), F32),
                   jax.ShapeDtypeStruct((n, h), F32)],
    )


def _tc_final(n, h, g, out, bn):
    grid = (n // bn,)
    fh = h // 2

    def body(pa_r, pb_r, d_r, r_r, bt_r, wfc_r, bfc_r, o_r, accp, accc):
        i = pl.program_id(0)

        @pl.when(i == 0)
        def _():
            accp[...] = jnp.zeros_like(accp)
            accc[...] = jnp.zeros_like(accc)

        inv = 1.0 / jnp.maximum(d_r[...][:, 0:1], 1.0)
        agg = jnp.concatenate([pa_r[...], pb_r[...]], axis=1)
        hb = jnp.maximum(agg * inv + r_r[...], 0.0)
        ids = bt_r[...]
        oh = (ids == lax.broadcasted_iota(jnp.int32, (bn, g), 1)).astype(F32)
        accp[...] += lax.dot_general(oh, hb, (((0,), (0,)), ((), ())),
                                     preferred_element_type=F32)
        accc[...] += jnp.sum(oh, axis=0)[:, None]

        @pl.when(i == pl.num_programs(0) - 1)
        def _():
            pooled = accp[...] / jnp.maximum(accc[...], 1.0)
            o_r[...] = jnp.dot(pooled, wfc_r[...],
                               preferred_element_type=F32) + bfc_r[...]

    return pl.pallas_call(
        body,
        grid=grid,
        in_specs=[pl.BlockSpec((bn, fh), lambda i: (i, 0)),
                  pl.BlockSpec((bn, fh), lambda i: (i, 0)),
                  pl.BlockSpec((bn, DW), lambda i: (i, 0)),
                  pl.BlockSpec((bn, h), lambda i: (i, 0)),
                  pl.BlockSpec((bn, 1), lambda i: (i, 0)),
                  pl.BlockSpec((h, out), lambda i: (0, 0)),
                  pl.BlockSpec((1, out), lambda i: (0, 0))],
        out_specs=pl.BlockSpec((g, out), lambda i: (0, 0)),
        out_shape=jax.ShapeDtypeStruct((g, out), F32),
        scratch_shapes=[pltpu.VMEM((g, h), F32), pltpu.VMEM((g, h), F32)],
    )


@functools.cache
def _build(n, e, fin, h, out, g):
    bn = 2000
    sc1 = _build_sc_agg(n, e, h, with_deg=True)
    sc23 = _build_sc_agg(n, e, h, with_deg=False)
    tc1 = _tc_first(n, fin, h, bn)
    tcm = _tc_mid(n, h, bn)
    tcf = _tc_final(n, h, g, out, bn)

    def run(x, edge_index, batch,
            wn1, wr1, b1, wn2, wr2, b2, wn3, wr3, b3, wfc, bfc):
        src = edge_index[0].astype(jnp.int32).reshape(-1, CH)
        dst = edge_index[1].astype(jnp.int32).reshape(-1, CH)
        z = jnp.zeros((n, h // 2), F32)
        z16 = jnp.zeros((n, DW), F32)
        ones = jnp.ones((CH, DW), F32)
        bt = batch.astype(jnp.int32)[:, None]

        ya, yb, r = tc1(x, wn1, wr1, b1[None, :])
        pa, pb, d = sc1(ya, yb, src, dst, z, z16, ones)
        ya, yb, r = tcm(pa, pb, d, r, wn2, wr2, b2[None, :])
        pa, pb = sc23(ya, yb, src, dst, z)
        ya, yb, r = tcm(pa, pb, d, r, wn3, wr3, b3[None, :])
        pa, pb = sc23(ya, yb, src, dst, z)
        return tcf(pa, pb, d, r, bt, wfc, bfc[None, :])

    return run


def kernel(x, edge_index, edge_attr, batch,
           W_neigh1, W_root1, b1, W_neigh2, W_root2, b2,
           W_neigh3, W_root3, b3, W_fc, b_fc):
    n, fin = x.shape
    e = edge_index.shape[1]
    h = W_neigh1.shape[1]
    out = W_fc.shape[1]
    g = 128
    run = _build(n, e, fin, h, out, g)
    return run(x, edge_index, batch,
               W_neigh1, W_root1, b1, W_neigh2, W_root2, b2,
               W_neigh3, W_root3, b3, W_fc, b_fc)
